# Initial kernel scaffold; baseline (speedup 1.0000x reference)
import jax
import jax.numpy as jnp
from jax.experimental import pallas as pl


def kernel(map_tensor, piece_tensor, edge_index, W1, W1s, b1, W2, W2s, b2, Wa, ba, Wb, bb, Wc, bc):
    raise NotImplementedError("write your pallas kernel here")


# speedup vs baseline: 1.7027x; 1.7027x over previous
import jax
import jax.numpy as jnp
from jax.experimental import pallas as pl
from jax.experimental.pallas import tpu as pltpu

_NEG = float(jnp.finfo(jnp.float32).min)


def _elu(v):
    return jnp.where(v > 0, v, jnp.exp(jnp.minimum(v, 0.0)) - 1.0)


def _head_body(ps_ref, mask_ref, gum_ref, Wa_ref, ba_ref, Wb_ref, bb_ref,
               Wc_ref, bc_ref, act_ref, lm_ref):
    ps = ps_ref[...]
    h = _elu(jnp.dot(ps, Wa_ref[...], preferred_element_type=jnp.float32)
                   + ba_ref[...])
    h = _elu(jnp.dot(h, Wb_ref[...], preferred_element_type=jnp.float32)
                   + bb_ref[...])
    logits = jnp.dot(h, Wc_ref[...], preferred_element_type=jnp.float32) + bc_ref[...]
    lm = jnp.where(mask_ref[...] != 0, logits, _NEG)
    lm_ref[...] = lm
    act_ref[...] = jnp.argmax(lm + gum_ref[...], axis=-1).astype(jnp.int32)[None, :]


def _head(piece_state, mask, gumbel, Wa, ba, Wb, bb, Wc, bc):
    B = piece_state.shape[0]
    act2d, lm = pl.pallas_call(
        _head_body,
        out_shape=(jax.ShapeDtypeStruct((1, B), jnp.int32),
                   jax.ShapeDtypeStruct((B, mask.shape[1]), jnp.float32)),
    )(piece_state, mask, gumbel, Wa, ba[None, :], Wb, bb[None, :], Wc, bc[None, :])
    return act2d[0], lm


def kernel(map_tensor, piece_tensor, edge_index, W1, W1s, b1, W2, W2s, b2,
           Wa, ba, Wb, bb, Wc, bc):
    batches = map_tensor.shape[0]
    features = map_tensor.shape[2]
    x = map_tensor.reshape(batches, -1, features)
    npg = x.shape[1]
    src = edge_index[0]
    dst = edge_index[1]

    agg = jnp.zeros_like(x).at[:, dst].add(x[:, src])
    h = jax.nn.elu(agg @ W1 + x @ W1s + b1)
    p_type = piece_tensor[:, 0].astype(jnp.int32)
    pos = piece_tensor[:, 1:3].astype(jnp.int32)
    action_mask = piece_tensor[:, 3:16].astype(bool)
    j = pos[:, 0] * 12 + pos[:, 1]

    sel = (dst[None, :] == j[:, None]).astype(jnp.float32)
    M = jnp.zeros((batches, npg), jnp.float32).at[:, src].add(sel)
    agg2_rows = jnp.einsum('bn,bnf->bf', M, h)
    hj = jnp.take_along_axis(h, j[:, None, None], axis=1)[:, 0, :]
    out_rows = jax.nn.elu(agg2_rows @ W2 + hj @ W2s + b2)

    p_onehot = jax.nn.one_hot(p_type, 3, dtype=jnp.float32)
    piece_state = jnp.concatenate([out_rows, p_onehot], axis=1)
    gumbel = jax.random.gumbel(jax.random.key(42), (batches, Wc.shape[1]),
                               jnp.float32)
    action, lm = _head(piece_state, action_mask.astype(jnp.int32), gumbel,
                       Wa, ba, Wb, bb, Wc, bc)
    return (action, lm)


# speedup vs baseline: 17.6818x; 10.3843x over previous
import functools

import jax
import jax.numpy as jnp
from jax import lax
from jax.experimental import pallas as pl
from jax.experimental.pallas import tpu as pltpu
from jax.experimental.pallas import tpu_sc as plsc

_NEG = float(jnp.finfo(jnp.float32).min)
_NC, _NS, _L = 2, 16, 16


def _elu(v):
    return jnp.where(v > 0, v, jnp.exp(jnp.minimum(v, 0.0)) - 1.0)


def _scatter_body(B, N, x_hbm, srcg_hbm, dst2_hbm, jsplat_hbm, z_hbm,
                  agg_hbm, m_hbm,
                  src_v, dst_v, rows_v, mpart, jv, idxrow_v, sem,
                  shared, sharedM):
    c = lax.axis_index("c")
    s = lax.axis_index("s")
    bpc = B // _NC
    rps = N // _NS
    lane = lax.iota(jnp.int32, _L)
    seven = jnp.full((_L,), 7, jnp.int32)
    low7 = jnp.full((_L,), 127, jnp.int32)
    one_vec = jnp.full((_L,), 1.0, jnp.float32)
    zero_vec = jnp.full((_L,), 0.0, jnp.float32)
    pltpu.sync_copy(dst2_hbm.at[pl.ds(s * 16, 16)], dst_v)
    for t in range(8):
        idxrow_v[pl.ds(t * 16, 16)] = lane

    def batch_body(i, carry):
        b = c * bpc + i
        pltpu.sync_copy(jsplat_hbm.at[b], jv)
        jb_vec = jv[...]
        bN_vec = jnp.full((_L,), b * N, jnp.int32)
        pltpu.sync_copy(z_hbm.at[pl.ds(s * rps, rps)],
                        shared.at[pl.ds(s * rps, rps)])
        pltpu.sync_copy(z_hbm.at[pl.ds(0, 256)], mpart)

        @pl.when(s == 0)
        def _():
            pltpu.sync_copy(z_hbm.at[pl.ds(0, 16)], sharedM)

        pltpu.sync_copy(srcg_hbm.at[pl.ds(b * 256 + s * 16, 16)], src_v)
        plsc.subcore_barrier()

        for k in range(16):
            pltpu.async_copy(x_hbm.at[src_v.at[k]], rows_v, sem).wait()
            pltpu.sync_copy(rows_v, shared.at[dst_v.at[k]], add=True)
            for q in range(8):
                sv = src_v[k, pl.ds(q * 16, 16)] - bN_vec
                dv = dst_v[k, pl.ds(q * 16, 16)]
                val = jnp.where(dv == jb_vec, one_vec, zero_vec)
                row = lane * 16 + lax.shift_right_logical(sv, seven)
                col = lax.bitwise_and(sv, low7)
                plsc.addupdate_scatter(mpart, [row, col], val)

        pltpu.sync_copy(mpart.at[pl.ds(0, 128)], sharedM.at[idxrow_v],
                        add=True)
        pltpu.sync_copy(mpart.at[pl.ds(128, 128)], sharedM.at[idxrow_v],
                        add=True)
        plsc.subcore_barrier()
        pltpu.sync_copy(shared.at[pl.ds(s * rps, rps)],
                        agg_hbm.at[pl.ds(b * N + s * rps, rps)])

        @pl.when(s == 0)
        def _():
            pltpu.sync_copy(sharedM, m_hbm.at[pl.ds(b * 16, 16)])

        plsc.subcore_barrier()
        return carry

    lax.fori_loop(0, bpc, batch_body, 0)


def _sc_scatter(x_flat, srcg, dst2, jsplat, zeros):
    B = 16
    N, F = zeros.shape
    mesh = plsc.VectorSubcoreMesh(core_axis_name="c", subcore_axis_name="s",
                                  num_cores=_NC, num_subcores=_NS)
    fn = pl.kernel(
        functools.partial(_scatter_body, B, N),
        out_type=(jax.ShapeDtypeStruct((B * N, F), jnp.float32),
                  jax.ShapeDtypeStruct((B * 16, 128), jnp.float32)),
        mesh=mesh,
        compiler_params=pltpu.CompilerParams(needs_layout_passes=False),
        scratch_types=[
            pltpu.VMEM((16, 128), jnp.int32),
            pltpu.VMEM((16, 128), jnp.int32),
            pltpu.VMEM((128, F), jnp.float32),
            pltpu.VMEM((256, 128), jnp.float32),
            pltpu.VMEM((16,), jnp.int32),
            pltpu.VMEM((128,), jnp.int32),
            pltpu.SemaphoreType.DMA,
            pltpu.VMEM_SHARED((N, F), jnp.float32),
            pltpu.VMEM_SHARED((16, 128), jnp.float32),
        ],
    )
    return fn(x_flat, srcg, dst2, jsplat, zeros)


def _mm1_body(MJ_ref, agg_ref, x_ref, W1_ref, W1s_ref, b1_ref, out_ref,
              h_scr):
    acc = jnp.dot(agg_ref[...], W1_ref[...],
                  preferred_element_type=jnp.float32)
    acc += jnp.dot(x_ref[...], W1s_ref[...],
                   preferred_element_type=jnp.float32)
    h_scr[...] = _elu(acc + b1_ref[...])
    out_ref[0] = jnp.dot(MJ_ref[0], h_scr[...],
                         preferred_element_type=jnp.float32)


def _tc_mm1(MJ, agg, x_flat, W1, W1s, b1):
    BN, F = x_flat.shape
    H = W1.shape[1]
    B = MJ.shape[0]
    blk = BN // B
    return pl.pallas_call(
        _mm1_body,
        grid=(B,),
        in_specs=[
            pl.BlockSpec((1, 2, blk), lambda b: (b, 0, 0)),
            pl.BlockSpec((blk, F), lambda b: (b, 0)),
            pl.BlockSpec((blk, F), lambda b: (b, 0)),
            pl.BlockSpec((F, H), lambda b: (0, 0)),
            pl.BlockSpec((F, H), lambda b: (0, 0)),
            pl.BlockSpec((1, H), lambda b: (0, 0)),
        ],
        out_specs=pl.BlockSpec((1, 2, H), lambda b: (b, 0, 0)),
        out_shape=jax.ShapeDtypeStruct((B, 2, H), jnp.float32),
        scratch_shapes=[pltpu.VMEM((blk, H), jnp.float32)],
    )(MJ, agg, x_flat, W1, W1s, b1[None, :])


def _head_body(rh_ref, W2_ref, W2s_ref, b2_ref, p3_ref, mask_ref,
               gum_ref, Wae_ref, Wap_ref, ba_ref, Wb_ref, bb_ref, Wc_ref,
               bc_ref, act_ref, lm_ref):
    r2 = rh_ref[:, 0, :]
    hj = rh_ref[:, 1, :]
    out_rows = _elu(
        jnp.dot(r2, W2_ref[...], preferred_element_type=jnp.float32)
        + jnp.dot(hj, W2s_ref[...], preferred_element_type=jnp.float32)
        + b2_ref[...])
    h = _elu(
        jnp.dot(out_rows, Wae_ref[...], preferred_element_type=jnp.float32)
        + jnp.dot(p3_ref[...], Wap_ref[...], preferred_element_type=jnp.float32)
        + ba_ref[...])
    h = _elu(jnp.dot(h, Wb_ref[...], preferred_element_type=jnp.float32)
             + bb_ref[...])
    logits = (jnp.dot(h, Wc_ref[...], preferred_element_type=jnp.float32)
              + bc_ref[...])
    lm = jnp.where(mask_ref[...] != 0, logits, _NEG)
    lm_ref[...] = lm
    act_ref[...] = jnp.argmax(lm + gum_ref[...], axis=-1).astype(jnp.int32)[None, :]


def _tc_head(rh, W2, W2s, b2, p3, mask, gumbel, Wae, Wap, ba, Wb, bb, Wc, bc):
    B = rh.shape[0]
    NA = Wc.shape[1]
    act2d, lm = pl.pallas_call(
        _head_body,
        out_shape=(jax.ShapeDtypeStruct((1, B), jnp.int32),
                   jax.ShapeDtypeStruct((B, NA), jnp.float32)),
    )(rh, W2, W2s, b2[None, :], p3, mask, gumbel,
      Wae, Wap, ba[None, :], Wb, bb[None, :], Wc, bc[None, :])
    return act2d[0], lm


def kernel(map_tensor, piece_tensor, edge_index, W1, W1s, b1, W2, W2s, b2,
           Wa, ba, Wb, bb, Wc, bc):
    B = map_tensor.shape[0]
    F = map_tensor.shape[2]
    x = map_tensor.reshape(B, -1, F)
    N = x.shape[1]
    x_flat = x.reshape(B * N, F)
    E = edge_index.shape[1]

    src = edge_index[0].astype(jnp.int32)
    dst = edge_index[1].astype(jnp.int32)
    offs = jnp.arange(B, dtype=jnp.int32) * N
    srcg = (src[None, :] + offs[:, None]).reshape(B * E // 128, 128)
    dst2 = dst.reshape(E // 128, 128)
    zeros = jnp.zeros((N, F), jnp.float32)

    p_type = piece_tensor[:, 0].astype(jnp.int32)
    pos = piece_tensor[:, 1:3].astype(jnp.int32)
    action_mask = piece_tensor[:, 3:16].astype(jnp.int32)
    j = pos[:, 0] * 12 + pos[:, 1]
    p3 = jax.nn.one_hot(p_type, 3, dtype=jnp.float32)
    gumbel = jax.random.gumbel(jax.random.key(42), (B, Wc.shape[1]),
                               jnp.float32)

    jsplat = jnp.broadcast_to(j[:, None], (B, 16)).astype(jnp.int32)
    agg, m_rows = _sc_scatter(x_flat, srcg, dst2, jsplat, zeros)
    M = m_rows.reshape(B, N)
    J = jax.nn.one_hot(j, N, dtype=jnp.float32)
    MJ = jnp.stack([M, J], axis=1)
    rh = _tc_mm1(MJ, agg, x_flat, W1, W1s, b1)
    action, lm = _tc_head(rh, W2, W2s, b2, p3, action_mask, gumbel,
                          Wa[:-3], Wa[-3:], ba, Wb, bb, Wc, bc)
    return (action, lm)


# speedup vs baseline: 39.8298x; 2.2526x over previous
import functools

import jax
import jax.numpy as jnp
from jax import lax
from jax.experimental import pallas as pl
from jax.experimental.pallas import tpu as pltpu
from jax.experimental.pallas import tpu_sc as plsc

_NEG = float(jnp.finfo(jnp.float32).min)
_NC, _NS, _L = 2, 16, 16


def _elu(v):
    return jnp.where(v > 0, v, jnp.exp(jnp.minimum(v, 0.0)) - 1.0)


def _build_body(N, src2_hbm, dst2_hbm, jsplat_hbm, za_hbm, zm_hbm,
                A_hbm, m_hbm,
                srcc, dstc, Abuf, mpart, jv, idxrow, sem, sharedM):
    c = lax.axis_index("c")
    s = lax.axis_index("s")
    wid = s * _NC + c
    lane = lax.iota(jnp.int32, _L)
    seven = jnp.full((_L,), 7, jnp.int32)
    low7 = jnp.full((_L,), 127, jnp.int32)
    one_vec = jnp.full((_L,), 1.0, jnp.float32)
    zero_vec = jnp.full((_L,), 0.0, jnp.float32)
    zero_ivec = jnp.full((_L,), 0, jnp.int32)
    r32u = jnp.full((_L,), 32, jnp.uint32)
    lane16 = lane * 16

    pltpu.sync_copy(jsplat_hbm.at[s], jv)
    cvec = jnp.full((_L,), c, jnp.int32)
    jb_vec = jnp.where(cvec == 0, jv[...], jnp.full((_L,), -1, jnp.int32))
    s16 = jnp.full((_L,), s * 16, jnp.int32)
    for t in range(8):
        idxrow[pl.ds(t * 16, 16)] = lane + s16

    @pl.when(jnp.logical_and(c == 0, s == 0))
    def _():
        pltpu.sync_copy(zm_hbm, sharedM)

    pltpu.sync_copy(zm_hbm, mpart)
    plsc.subcore_barrier()

    for p in range(2):
        row0 = wid * 64 + p * 32
        lo_vec = jnp.full((_L,), row0, jnp.int32)
        pltpu.sync_copy(za_hbm, Abuf)
        for ch in range(4):
            pltpu.sync_copy(src2_hbm.at[pl.ds(ch * 64, 64)], srcc)
            pltpu.sync_copy(dst2_hbm.at[pl.ds(ch * 64, 64)], dstc)

            def scanrow(r, carry):
                for q in range(8):
                    sv = srcc[r, pl.ds(q * 16, 16)]
                    dv = dstc[r, pl.ds(q * 16, 16)]
                    u = dv - lo_vec
                    m = plsc.bitcast(u, jnp.uint32) < r32u
                    rowi = jnp.where(m, u, zero_ivec)
                    val = jnp.where(m, one_vec, zero_vec)
                    plsc.addupdate_scatter(Abuf, [rowi, sv], val)
                    if p == 0:
                        valm = jnp.where(dv == jb_vec, one_vec, zero_vec)
                        rowm = lane16 + lax.shift_right_logical(sv, seven)
                        colm = lax.bitwise_and(sv, low7)
                        plsc.addupdate_scatter(mpart, [rowm, colm], valm)
                return carry

            lax.fori_loop(0, 64, scanrow, 0)
        pltpu.sync_copy(Abuf, A_hbm.at[pl.ds(row0, 32)])

    @pl.when(c == 0)
    def _():
        pltpu.sync_copy(mpart.at[pl.ds(0, 128)], sharedM.at[idxrow],
                        add=True)
        pltpu.sync_copy(mpart.at[pl.ds(128, 128)], sharedM.at[idxrow],
                        add=True)

    plsc.subcore_barrier()

    @pl.when(jnp.logical_and(c == 0, s == 0))
    def _():
        pltpu.sync_copy(sharedM, m_hbm)


def _sc_build(src2, dst2, jsplat, za, zm):
    N = 2048
    mesh = plsc.VectorSubcoreMesh(core_axis_name="c", subcore_axis_name="s",
                                  num_cores=_NC, num_subcores=_NS)
    fn = pl.kernel(
        functools.partial(_build_body, N),
        out_type=(jax.ShapeDtypeStruct((N, N), jnp.float32),
                  jax.ShapeDtypeStruct((256, 128), jnp.float32)),
        mesh=mesh,
        compiler_params=pltpu.CompilerParams(needs_layout_passes=False),
        scratch_types=[
            pltpu.VMEM((64, 128), jnp.int32),
            pltpu.VMEM((64, 128), jnp.int32),
            pltpu.VMEM((32, 2048), jnp.float32),
            pltpu.VMEM((256, 128), jnp.float32),
            pltpu.VMEM((16,), jnp.int32),
            pltpu.VMEM((128,), jnp.int32),
            pltpu.SemaphoreType.DMA,
            pltpu.VMEM_SHARED((256, 128), jnp.float32),
        ],
    )
    return fn(src2, dst2, jsplat, za, zm)


def _mm_body(A_ref, x_ref, MJ_ref, W1_ref, W1s_ref, b1_ref, out_ref, h_scr):
    agg = jnp.dot(A_ref[...], x_ref[...], preferred_element_type=jnp.float32)
    acc = jnp.dot(agg, W1_ref[...], preferred_element_type=jnp.float32)
    acc += jnp.dot(x_ref[...], W1s_ref[...],
                   preferred_element_type=jnp.float32)
    h_scr[...] = _elu(acc + b1_ref[...])
    out_ref[0] = jnp.dot(MJ_ref[0], h_scr[...],
                         preferred_element_type=jnp.float32)


def _tc_mm(A, x_flat, MJ, W1, W1s, b1):
    BN, F = x_flat.shape
    H = W1.shape[1]
    B = MJ.shape[0]
    blk = BN // B
    return pl.pallas_call(
        _mm_body,
        grid=(B,),
        in_specs=[
            pl.BlockSpec((blk, blk), lambda b: (0, 0)),
            pl.BlockSpec((blk, F), lambda b: (b, 0)),
            pl.BlockSpec((1, 2, blk), lambda b: (b, 0, 0)),
            pl.BlockSpec((F, H), lambda b: (0, 0)),
            pl.BlockSpec((F, H), lambda b: (0, 0)),
            pl.BlockSpec((1, H), lambda b: (0, 0)),
        ],
        out_specs=pl.BlockSpec((1, 2, H), lambda b: (b, 0, 0)),
        out_shape=jax.ShapeDtypeStruct((B, 2, H), jnp.float32),
        scratch_shapes=[pltpu.VMEM((blk, H), jnp.float32)],
        compiler_params=pltpu.CompilerParams(
            vmem_limit_bytes=56 * 1024 * 1024),
    )(A, x_flat, MJ, W1, W1s, b1[None, :])


def _head_body(rh_ref, W2_ref, W2s_ref, b2_ref, p3_ref, mask_ref,
               gum_ref, Wae_ref, Wap_ref, ba_ref, Wb_ref, bb_ref, Wc_ref,
               bc_ref, act_ref, lm_ref):
    r2 = rh_ref[:, 0, :]
    hj = rh_ref[:, 1, :]
    out_rows = _elu(
        jnp.dot(r2, W2_ref[...], preferred_element_type=jnp.float32)
        + jnp.dot(hj, W2s_ref[...], preferred_element_type=jnp.float32)
        + b2_ref[...])
    h = _elu(
        jnp.dot(out_rows, Wae_ref[...], preferred_element_type=jnp.float32)
        + jnp.dot(p3_ref[...], Wap_ref[...], preferred_element_type=jnp.float32)
        + ba_ref[...])
    h = _elu(jnp.dot(h, Wb_ref[...], preferred_element_type=jnp.float32)
             + bb_ref[...])
    logits = (jnp.dot(h, Wc_ref[...], preferred_element_type=jnp.float32)
              + bc_ref[...])
    lm = jnp.where(mask_ref[...] != 0, logits, _NEG)
    lm_ref[...] = lm
    act_ref[...] = jnp.argmax(lm + gum_ref[...], axis=-1).astype(jnp.int32)[None, :]


def _tc_head(rh, W2, W2s, b2, p3, mask, gumbel, Wae, Wap, ba, Wb, bb, Wc, bc):
    B = rh.shape[0]
    NA = Wc.shape[1]
    act2d, lm = pl.pallas_call(
        _head_body,
        out_shape=(jax.ShapeDtypeStruct((1, B), jnp.int32),
                   jax.ShapeDtypeStruct((B, NA), jnp.float32)),
    )(rh, W2, W2s, b2[None, :], p3, mask, gumbel,
      Wae, Wap, ba[None, :], Wb, bb[None, :], Wc, bc[None, :])
    return act2d[0], lm


def kernel(map_tensor, piece_tensor, edge_index, W1, W1s, b1, W2, W2s, b2,
           Wa, ba, Wb, bb, Wc, bc):
    B = map_tensor.shape[0]
    F = map_tensor.shape[2]
    x = map_tensor.reshape(B, -1, F)
    N = x.shape[1]
    x_flat = x.reshape(B * N, F)
    E = edge_index.shape[1]

    src = edge_index[0].astype(jnp.int32)
    dst = edge_index[1].astype(jnp.int32)
    src2 = src.reshape(E // 128, 128)
    dst2 = dst.reshape(E // 128, 128)
    za = jnp.zeros((32, N), jnp.float32)
    zm = jnp.zeros((256, 128), jnp.float32)

    p_type = piece_tensor[:, 0].astype(jnp.int32)
    pos = piece_tensor[:, 1:3].astype(jnp.int32)
    action_mask = piece_tensor[:, 3:16].astype(jnp.int32)
    j = pos[:, 0] * 12 + pos[:, 1]
    jsplat = jnp.broadcast_to(j[:, None], (B, 16)).astype(jnp.int32)
    p3 = jax.nn.one_hot(p_type, 3, dtype=jnp.float32)
    gumbel = jax.random.gumbel(jax.random.key(42), (B, Wc.shape[1]),
                               jnp.float32)

    A, m_rows = _sc_build(src2, dst2, jsplat, za, zm)
    M = m_rows.reshape(B, N)
    J = jax.nn.one_hot(j, N, dtype=jnp.float32)
    MJ = jnp.stack([M, J], axis=1)
    rh = _tc_mm(A, x_flat, MJ, W1, W1s, b1)
    action, lm = _tc_head(rh, W2, W2s, b2, p3, action_mask, gumbel,
                          Wa[:-3], Wa[-3:], ba, Wb, bb, Wc, bc)
    return (action, lm)
